# baseline (speedup 1.0000x reference)
import functools

import jax
import jax.numpy as jnp
from jax import lax
from jax.experimental import pallas as pl
from jax.experimental.pallas import tpu as pltpu
from jax.experimental.pallas import tpu_sc as plsc

_CHUNK = 128


@functools.lru_cache(maxsize=None)
def _make_sc_gather(n_chunks, v4):
    info = plsc.get_sparse_core_info()
    nc, ns = info.num_cores, info.num_subcores
    nw = nc * ns
    per_w = n_chunks // nw

    mesh = plsc.VectorSubcoreMesh(core_axis_name="c", subcore_axis_name="s")

    @functools.partial(
        pl.kernel,
        mesh=mesh,
        out_type=jax.ShapeDtypeStruct((n_chunks, _CHUNK, 128), jnp.float32),
        scratch_types=[
            pltpu.VMEM((per_w, _CHUNK), jnp.int32),
            pltpu.VMEM((per_w, _CHUNK, 128), jnp.float32),
            pltpu.SemaphoreType.DMA,
        ],
    )
    def gather_kernel(tok_hbm, idx_hbm, out_hbm, idx_v, rows_v, sem):
        wid = lax.axis_index("s") * nc + lax.axis_index("c")
        base = wid * per_w
        pltpu.sync_copy(idx_hbm.at[wid], idx_v)
        copies = [
            pltpu.async_copy(tok_hbm.at[idx_v.at[k]], rows_v.at[k], sem)
            for k in range(per_w)
        ]
        for c in copies:
            c.wait()
        pltpu.sync_copy(rows_v, out_hbm.at[pl.ds(base, per_w)])

    return gather_kernel


def _matmul_body(x4_ref, sub_ref, w_ref, b_ref, pos_ref, o_ref, *, bt, t, vt):
    w = w_ref[:].astype(jnp.bfloat16)
    sub = sub_ref[:]
    x4 = x4_ref[:]
    x = jnp.where(sub == 0, x4[:, 0:32], 0.0)
    for s in range(1, 4):
        x = x + jnp.where(sub == s, x4[:, s * 32:(s + 1) * 32], 0.0)
    acc = jnp.dot(x.astype(jnp.bfloat16), w, preferred_element_type=jnp.float32)
    p = jnp.dot(pos_ref[:].astype(jnp.bfloat16), w,
                preferred_element_type=jnp.float32) + b_ref[:]
    pt = jnp.broadcast_to(p[None, :, :], (bt // t, t, vt)).reshape(bt, vt)
    o_ref[:] = acc + pt


def _lm_head(x4, sub, W, b2, pos_table, *, bt, vt):
    bf = x4.shape[0]
    d = W.shape[0]
    t = pos_table.shape[0]
    v = W.shape[1]
    grid = (bf // bt, pl.cdiv(v, vt))
    return pl.pallas_call(
        functools.partial(_matmul_body, bt=bt, t=t, vt=vt),
        grid=grid,
        in_specs=[
            pl.BlockSpec((bt, 128), lambda i, j: (i, 0)),
            pl.BlockSpec((bt, 1), lambda i, j: (i, 0)),
            pl.BlockSpec((d, vt), lambda i, j: (0, j)),
            pl.BlockSpec((1, vt), lambda i, j: (0, j)),
            pl.BlockSpec((t, d), lambda i, j: (0, 0)),
        ],
        out_specs=pl.BlockSpec((bt, vt), lambda i, j: (i, j)),
        out_shape=jax.ShapeDtypeStruct((bf, v), jnp.float32),
        compiler_params=pltpu.CompilerParams(
            dimension_semantics=("parallel", "parallel"),
        ),
    )(x4, sub, W, b2, pos_table)


def kernel(idx, tok_table, pos_table, W, b):
    B, T = idx.shape
    V, D = tok_table.shape
    bf = B * T
    n_chunks = bf // _CHUNK
    nw = 32
    idxf = idx.reshape(bf).astype(jnp.int32)
    tok2 = tok_table.reshape(V // 4, 128)
    idx4 = (idxf // 4).reshape(nw, n_chunks // nw, _CHUNK)
    sub = (idxf % 4).reshape(bf, 1)
    x4 = _make_sc_gather(n_chunks, V // 4)(tok2, idx4)
    x4 = x4.reshape(bf, 128)
    out2 = _lm_head(x4, sub, W, b.reshape(1, V), pos_table, bt=1024, vt=2048)
    return out2.reshape(B, T, V)


# speedup vs baseline: 3.0344x; 3.0344x over previous
import functools

import jax
import jax.numpy as jnp
from jax import lax
from jax.experimental import pallas as pl
from jax.experimental.pallas import tpu as pltpu
from jax.experimental.pallas import tpu_sc as plsc

_CHUNK = 128


@functools.lru_cache(maxsize=None)
def _make_sc_gather(n_chunks, v4):
    info = plsc.get_sparse_core_info()
    nc, ns = info.num_cores, info.num_subcores
    nw = nc * ns
    per_w = n_chunks // nw

    mesh = plsc.VectorSubcoreMesh(core_axis_name="c", subcore_axis_name="s")

    @functools.partial(
        pl.kernel,
        mesh=mesh,
        out_type=jax.ShapeDtypeStruct((n_chunks, _CHUNK, 128), jnp.float32),
        scratch_types=[
            pltpu.VMEM((per_w, _CHUNK), jnp.int32),
            pltpu.VMEM((per_w, _CHUNK, 128), jnp.float32),
            pltpu.SemaphoreType.DMA,
        ],
    )
    def gather_kernel(tok_hbm, idx_hbm, out_hbm, idx_v, rows_v, sem):
        wid = lax.axis_index("s") * nc + lax.axis_index("c")
        base = wid * per_w
        pltpu.sync_copy(idx_hbm.at[wid], idx_v)
        copies = [
            pltpu.async_copy(tok_hbm.at[idx_v.at[k]], rows_v.at[k], sem)
            for k in range(per_w)
        ]
        for c in copies:
            c.wait()
        pltpu.sync_copy(rows_v, out_hbm.at[pl.ds(base, per_w)])

    return gather_kernel


def _matmul_body(x4_ref, sub_ref, waug_ref, pos_ref, o_ref, *, bb, d, vt):
    x4 = x4_ref[0]
    sub = sub_ref[0, :, 0:d]
    x = jnp.where(sub == 0.0, x4[:, 0:d], 0.0)
    for s in range(1, 4):
        x = x + jnp.where(sub == float(s), x4[:, s * d:(s + 1) * d], 0.0)
    x = x + pos_ref[0]
    xaug = jnp.concatenate(
        [x, jnp.ones((bb, 1), jnp.float32)], axis=1).astype(jnp.bfloat16)
    w = waug_ref[:].astype(jnp.bfloat16)
    o_ref[0] = lax.dot_general(
        w, xaug, (((0,), (1,)), ((), ())),
        preferred_element_type=jnp.float32)


def _lm_head(x4, subf, Waug, pos_table, *, vt):
    t, bb, _ = x4.shape
    d = pos_table.shape[2]
    v = Waug.shape[1]
    grid = (t, pl.cdiv(v, vt))
    return pl.pallas_call(
        functools.partial(_matmul_body, bb=bb, d=d, vt=vt),
        grid=grid,
        in_specs=[
            pl.BlockSpec((1, bb, 128), lambda i, j: (i, 0, 0)),
            pl.BlockSpec((1, bb, 128), lambda i, j: (i, 0, 0)),
            pl.BlockSpec((d + 1, vt), lambda i, j: (0, j)),
            pl.BlockSpec((1, 1, d), lambda i, j: (i, 0, 0)),
        ],
        out_specs=pl.BlockSpec((1, vt, bb), lambda i, j: (i, j, 0)),
        out_shape=jax.ShapeDtypeStruct((t, v, bb), jnp.float32),
        compiler_params=pltpu.CompilerParams(
            dimension_semantics=("parallel", "parallel"),
        ),
    )(x4, subf, Waug, pos_table)


def kernel(idx, tok_table, pos_table, W, b):
    B, T = idx.shape
    V, D = tok_table.shape
    bf = B * T
    n_chunks = bf // _CHUNK
    nw = 32
    idxT = idx.T.astype(jnp.int32)
    idxPf = idxT.reshape(bf)
    tok2 = tok_table.reshape(V // 4, 128)
    idx4 = (idxPf // 4).reshape(nw, n_chunks // nw, _CHUNK)
    subf = jnp.broadcast_to(
        (idxT % 4).astype(jnp.float32)[:, :, None], (T, B, 128))
    x4 = _make_sc_gather(n_chunks, V // 4)(tok2, idx4)
    x4 = x4.reshape(T, B, 128)
    Waug = jnp.concatenate([W, b[None, :]], axis=0)
    pos3 = pos_table.reshape(T, 1, D)
    outT = _lm_head(x4, subf, Waug, pos3, vt=2048)
    return jnp.transpose(outT, (2, 0, 1))
